# baseline (speedup 1.0000x reference)
import jax
import jax.numpy as jnp
from jax import lax
from jax.experimental import pallas as pl
from jax.experimental.pallas import tpu as pltpu
from jax.experimental.pallas import tpu_sc as plsc

T, N, D, H, G, OUT = 4, 10000, 128, 128, 16, 128
NC, NS = 2, 16
NW = NC * NS
CH = 128
N_PAD = 10240
TRASH = N
ZROWS = N_PAD // NS
CPW = 80
HCPW = CPW // 2


def _front_body(x_ref, wnt_ref, bnt_ref, wat_ref, bat_ref, h_ref, aw_ref):
    x = x_ref[...]
    wnt = wnt_ref[...]
    wat = wat_ref[...]
    bnt = bnt_ref[...]
    bat = bat_ref[...]
    ats = []
    for t in range(T):
        ht = jnp.dot(x[t], wnt, preferred_element_type=jnp.float32) + bnt
        h_ref[t] = ht
        ats.append(jnp.dot(ht, wat, preferred_element_type=jnp.float32) + bat)
    for i in range(T - 1):
        sc = ats[i] * ats[i + 1]
        m = jnp.max(sc, axis=-1, keepdims=True)
        e = jnp.exp(sc - m)
        aw_ref[i] = e / jnp.sum(e, axis=-1, keepdims=True)


def _tc_front(x, w_nt, b_nt, w_attn, b_attn):
    nb = 400
    grid = (N // nb,)
    return pl.pallas_call(
        _front_body,
        grid=grid,
        in_specs=[
            pl.BlockSpec((T, nb, D), lambda n: (0, n, 0)),
            pl.BlockSpec((D, H), lambda n: (0, 0)),
            pl.BlockSpec((1, H), lambda n: (0, 0)),
            pl.BlockSpec((H, H), lambda n: (0, 0)),
            pl.BlockSpec((1, H), lambda n: (0, 0)),
        ],
        out_specs=[
            pl.BlockSpec((T, nb, H), lambda n: (0, n, 0)),
            pl.BlockSpec((T - 1, nb, H), lambda n: (0, n, 0)),
        ],
        out_shape=[
            jax.ShapeDtypeStruct((T, N, H), jnp.float32),
            jax.ShapeDtypeStruct((T - 1, N, H), jnp.float32),
        ],
    )(x, w_nt, b_nt.reshape(1, H), w_attn, b_attn.reshape(1, H))


def _seg_body(aw_ref, dst_ref, src_ref, zeros_ref, parts_ref,
              ia, ib, sa, sb, rowsa, rowsb, acc, ga_sem, gb_sem):
    c = lax.axis_index("c")
    s = lax.axis_index("s")
    wid = s * NC + c
    e_pad = dst_ref.shape[0] // (T - 1)

    for i in range(T - 1):
        pltpu.sync_copy(zeros_ref, acc.at[pl.ds(s * ZROWS, ZROWS)])
        plsc.subcore_barrier()

        def pairstep(p, carry):
            ca = (wid * CPW + 2 * p) * CH
            cb = ca + CH
            pltpu.sync_copy(dst_ref.at[pl.ds(i * e_pad + ca, CH)], ia)
            g_a = pltpu.async_copy(aw_ref.at[ia], rowsa, ga_sem)
            pltpu.sync_copy(dst_ref.at[pl.ds(i * e_pad + cb, CH)], ib)
            g_b = pltpu.async_copy(aw_ref.at[ib], rowsb, gb_sem)
            pltpu.sync_copy(src_ref.at[pl.ds(ca, CH)], sa)
            pltpu.sync_copy(src_ref.at[pl.ds(cb, CH)], sb)
            g_a.wait()
            pltpu.sync_copy(rowsa, acc.at[sa], add=True)
            g_b.wait()
            pltpu.sync_copy(rowsb, acc.at[sb], add=True)
            return carry

        lax.fori_loop(0, CPW // 2, pairstep, 0)
        plsc.subcore_barrier()
        pltpu.sync_copy(acc.at[pl.ds(s * ZROWS, ZROWS)],
                        parts_ref.at[c, i, pl.ds(s * ZROWS, ZROWS)])
        plsc.subcore_barrier()


def _sc_segsum(aw_flat, dst1d, src1d, zeros):
    mesh = plsc.VectorSubcoreMesh(
        core_axis_name="c", subcore_axis_name="s", num_cores=NC,
        num_subcores=NS)
    return pl.kernel(
        _seg_body,
        out_type=jax.ShapeDtypeStruct((NC, T - 1, N_PAD, H), jnp.float32),
        mesh=mesh,
        scratch_types=[
            pltpu.VMEM((CH,), jnp.int32),
            pltpu.VMEM((CH,), jnp.int32),
            pltpu.VMEM((CH,), jnp.int32),
            pltpu.VMEM((CH,), jnp.int32),
            pltpu.VMEM((CH, H), jnp.float32),
            pltpu.VMEM((CH, H), jnp.float32),
            pltpu.VMEM_SHARED((N_PAD, H), jnp.float32),
            pltpu.SemaphoreType.DMA,
            pltpu.SemaphoreType.DMA,
        ],
    )(aw_flat, dst1d, src1d, zeros)


def _back_body(h_ref, parts_ref, bidx_ref, wsda_ref, wsdb_ref, bsd_ref,
               wg1_ref, bg1_ref, wg2_ref, bg2_ref, wm1_ref, bm1_ref,
               wm2_ref, bm2_ref, out_ref, acc):
    n = pl.program_id(0)
    nsteps = pl.num_programs(0)

    @pl.when(n == 0)
    def _():
        acc[...] = jnp.zeros_like(acc)

    h = h_ref[...]
    segs = parts_ref[0] + parts_ref[1]
    bsd = bsd_ref[...]
    tot = None
    for i in range(T - 1):
        w = (h[i + 1] - h[i]) * segs[i]
        e = jax.nn.relu(
            jnp.dot(h[i], wsda_ref[...], preferred_element_type=jnp.float32)
            + jnp.dot(w, wsdb_ref[...], preferred_element_type=jnp.float32)
            + bsd)
        tot = e if tot is None else tot + e
    prop = tot * (1.0 / (T - 1))
    g1 = jax.nn.relu(
        jnp.dot(prop, wg1_ref[...], preferred_element_type=jnp.float32)
        + bg1_ref[...])
    gate = jax.nn.sigmoid(
        jnp.sum(g1 * wg2_ref[...], axis=-1, keepdims=True) + bg2_ref[0, 0])
    gp = gate * prop
    bidx = bidx_ref[0, 0]
    mask = (lax.broadcasted_iota(jnp.int32, (G, gp.shape[0]), 0)
            == bidx[None, :]).astype(jnp.float32)
    acc[...] += jnp.dot(mask, gp, preferred_element_type=jnp.float32)

    @pl.when(n == nsteps - 1)
    def _():
        ge = acc[...]
        o = jax.nn.relu(
            jnp.dot(ge, wm1_ref[...], preferred_element_type=jnp.float32)
            + bm1_ref[...])
        out_ref[...] = (
            jnp.dot(o, wm2_ref[...], preferred_element_type=jnp.float32)
            + bm2_ref[...])


def _tc_back(h, parts, bidx3, w_sd, b_sd, w_g1, b_g1, w_g2, b_g2,
             w_m1, b_m1, w_m2, b_m2):
    nb = 400
    nblk = N // nb
    return pl.pallas_call(
        _back_body,
        grid=(nblk,),
        in_specs=[
            pl.BlockSpec((T, nb, H), lambda n: (0, n, 0)),
            pl.BlockSpec((NC, T - 1, nb, H), lambda n: (0, 0, n, 0)),
            pl.BlockSpec((1, 1, nb), lambda n: (n, 0, 0)),
            pl.BlockSpec((H, H), lambda n: (0, 0)),
            pl.BlockSpec((H, H), lambda n: (0, 0)),
            pl.BlockSpec((1, H), lambda n: (0, 0)),
            pl.BlockSpec((H, H), lambda n: (0, 0)),
            pl.BlockSpec((1, H), lambda n: (0, 0)),
            pl.BlockSpec((1, H), lambda n: (0, 0)),
            pl.BlockSpec((1, 1), lambda n: (0, 0)),
            pl.BlockSpec((H, H), lambda n: (0, 0)),
            pl.BlockSpec((1, H), lambda n: (0, 0)),
            pl.BlockSpec((H, OUT), lambda n: (0, 0)),
            pl.BlockSpec((1, OUT), lambda n: (0, 0)),
        ],
        out_specs=pl.BlockSpec((G, OUT), lambda n: (0, 0)),
        out_shape=jax.ShapeDtypeStruct((G, OUT), jnp.float32),
        scratch_shapes=[pltpu.VMEM((G, H), jnp.float32)],
    )(h, parts, bidx3, w_sd[:H], w_sd[H:], b_sd.reshape(1, H),
      w_g1, b_g1.reshape(1, H), w_g2.reshape(1, H), b_g2.reshape(1, 1),
      w_m1, b_m1.reshape(1, H), w_m2, b_m2.reshape(1, OUT))


def kernel(x, edge_index, batch_idx, W_nt, b_nt, W_attn, b_attn, W_sd, b_sd,
           W_g1, b_g1, W_g2, b_g2, W_m1, b_m1, W_m2, b_m2):
    src = edge_index[0].astype(jnp.int32)
    dst = edge_index[1].astype(jnp.int32)
    e = src.shape[0]
    grp = NW * CPW * CH
    e_pad = ((e + grp - 1) // grp) * grp
    src_pad = jnp.concatenate(
        [src, jnp.full((e_pad - e,), TRASH, jnp.int32)])
    dst_pad = jnp.concatenate([dst, jnp.zeros((e_pad - e,), jnp.int32)])
    dst_all = (dst_pad[None, :]
               + (jnp.arange(T - 1, dtype=jnp.int32) * N)[:, None])
    dst1d = dst_all.reshape(-1)
    zeros = jnp.zeros((ZROWS, H), jnp.float32)

    h, aw = _tc_front(x, W_nt, b_nt, W_attn, b_attn)
    parts = _sc_segsum(aw.reshape((T - 1) * N, H), dst1d, src_pad, zeros)
    return _tc_back(h, parts, batch_idx.astype(jnp.int32).reshape(N // 400, 1, 400),
                    W_sd, b_sd, W_g1, b_g1, W_g2, b_g2, W_m1, b_m1, W_m2, b_m2)


# speedup vs baseline: 1.2605x; 1.2605x over previous
import jax
import jax.numpy as jnp
from jax import lax
from jax.experimental import pallas as pl
from jax.experimental.pallas import tpu as pltpu
from jax.experimental.pallas import tpu_sc as plsc

T, N, D, H, G, OUT = 4, 10000, 128, 128, 16, 128
NC, NS = 2, 16
NW = NC * NS
CH = 128
N_PAD = 10240
TRASH = N
ZROWS = N_PAD // NS


def _front_body(x_ref, wnt_ref, bnt_ref, wat_ref, bat_ref, h_ref, aw_ref):
    x = x_ref[...]
    wnt = wnt_ref[...]
    wat = wat_ref[...]
    bnt = bnt_ref[...]
    bat = bat_ref[...]
    ats = []
    for t in range(T):
        ht = jnp.dot(x[t], wnt, preferred_element_type=jnp.float32) + bnt
        h_ref[t] = ht
        ats.append(jnp.dot(ht, wat, preferred_element_type=jnp.float32) + bat)
    for i in range(T - 1):
        sc = ats[i] * ats[i + 1]
        m = jnp.max(sc, axis=-1, keepdims=True)
        e = jnp.exp(sc - m)
        aw_ref[i] = e / jnp.sum(e, axis=-1, keepdims=True)


def _tc_front(x, w_nt, b_nt, w_attn, b_attn):
    nb = 400
    grid = (N // nb,)
    return pl.pallas_call(
        _front_body,
        grid=grid,
        in_specs=[
            pl.BlockSpec((T, nb, D), lambda n: (0, n, 0)),
            pl.BlockSpec((D, H), lambda n: (0, 0)),
            pl.BlockSpec((1, H), lambda n: (0, 0)),
            pl.BlockSpec((H, H), lambda n: (0, 0)),
            pl.BlockSpec((1, H), lambda n: (0, 0)),
        ],
        out_specs=[
            pl.BlockSpec((T, nb, H), lambda n: (0, n, 0)),
            pl.BlockSpec((T - 1, nb, H), lambda n: (0, n, 0)),
        ],
        out_shape=[
            jax.ShapeDtypeStruct((T, N, H), jnp.float32),
            jax.ShapeDtypeStruct((T - 1, N, H), jnp.float32),
        ],
    )(x, w_nt, b_nt.reshape(1, H), w_attn, b_attn.reshape(1, H))


def _seg_body(aw_ref, dst_ref, src_ref, zeros_ref, parts_ref,
              idx_dst, idx_src, rows, acc, sem):
    c = lax.axis_index("c")
    s = lax.axis_index("s")
    wid = s * NC + c
    e_pad = dst_ref.shape[0] // (T - 1)
    chunks_per_worker = e_pad // CH // NW

    for i in range(T - 1):
        pltpu.sync_copy(zeros_ref, acc.at[pl.ds(s * ZROWS, ZROWS)])
        plsc.subcore_barrier()

        def chunk_step(k, carry):
            base = (wid * chunks_per_worker + k) * CH
            pltpu.sync_copy(dst_ref.at[pl.ds(i * e_pad + base, CH)], idx_dst)
            pltpu.sync_copy(src_ref.at[pl.ds(base, CH)], idx_src)
            pltpu.async_copy(aw_ref.at[idx_dst], rows, sem).wait()
            pltpu.sync_copy(rows, acc.at[idx_src], add=True)
            return carry

        lax.fori_loop(0, chunks_per_worker, chunk_step, None)
        plsc.subcore_barrier()
        pltpu.sync_copy(acc.at[pl.ds(s * ZROWS, ZROWS)],
                        parts_ref.at[c, i, pl.ds(s * ZROWS, ZROWS)])
        plsc.subcore_barrier()


def _sc_segsum(aw_flat, dst1d, src1d, zeros):
    mesh = plsc.VectorSubcoreMesh(
        core_axis_name="c", subcore_axis_name="s", num_cores=NC,
        num_subcores=NS)
    return pl.kernel(
        _seg_body,
        out_type=jax.ShapeDtypeStruct((NC, T - 1, N_PAD, H), jnp.float32),
        mesh=mesh,
        scratch_types=[
            pltpu.VMEM((CH,), jnp.int32),
            pltpu.VMEM((CH,), jnp.int32),
            pltpu.VMEM((CH, H), jnp.float32),
            pltpu.VMEM_SHARED((N_PAD, H), jnp.float32),
            pltpu.SemaphoreType.DMA,
        ],
    )(aw_flat, dst1d, src1d, zeros)


def _back_body(h_ref, parts_ref, bidx_ref, wsda_ref, wsdb_ref, bsd_ref,
               wg1_ref, bg1_ref, wg2_ref, bg2_ref, wm1_ref, bm1_ref,
               wm2_ref, bm2_ref, out_ref, acc):
    n = pl.program_id(0)
    nsteps = pl.num_programs(0)

    @pl.when(n == 0)
    def _():
        acc[...] = jnp.zeros_like(acc)

    h = h_ref[...]
    segs = parts_ref[0] + parts_ref[1]
    bsd = bsd_ref[...]
    tot = None
    for i in range(T - 1):
        w = (h[i + 1] - h[i]) * segs[i]
        e = jax.nn.relu(
            jnp.dot(h[i], wsda_ref[...], preferred_element_type=jnp.float32)
            + jnp.dot(w, wsdb_ref[...], preferred_element_type=jnp.float32)
            + bsd)
        tot = e if tot is None else tot + e
    prop = tot * (1.0 / (T - 1))
    g1 = jax.nn.relu(
        jnp.dot(prop, wg1_ref[...], preferred_element_type=jnp.float32)
        + bg1_ref[...])
    gate = jax.nn.sigmoid(
        jnp.sum(g1 * wg2_ref[...], axis=-1, keepdims=True) + bg2_ref[0, 0])
    gp = gate * prop
    bidx = bidx_ref[0, 0]
    mask = (lax.broadcasted_iota(jnp.int32, (G, gp.shape[0]), 0)
            == bidx[None, :]).astype(jnp.float32)
    acc[...] += jnp.dot(mask, gp, preferred_element_type=jnp.float32)

    @pl.when(n == nsteps - 1)
    def _():
        ge = acc[...]
        o = jax.nn.relu(
            jnp.dot(ge, wm1_ref[...], preferred_element_type=jnp.float32)
            + bm1_ref[...])
        out_ref[...] = (
            jnp.dot(o, wm2_ref[...], preferred_element_type=jnp.float32)
            + bm2_ref[...])


def _tc_back(h, parts, bidx3, w_sd, b_sd, w_g1, b_g1, w_g2, b_g2,
             w_m1, b_m1, w_m2, b_m2):
    nb = 400
    nblk = N // nb
    return pl.pallas_call(
        _back_body,
        grid=(nblk,),
        in_specs=[
            pl.BlockSpec((T, nb, H), lambda n: (0, n, 0)),
            pl.BlockSpec((NC, T - 1, nb, H), lambda n: (0, 0, n, 0)),
            pl.BlockSpec((1, 1, nb), lambda n: (n, 0, 0)),
            pl.BlockSpec((H, H), lambda n: (0, 0)),
            pl.BlockSpec((H, H), lambda n: (0, 0)),
            pl.BlockSpec((1, H), lambda n: (0, 0)),
            pl.BlockSpec((H, H), lambda n: (0, 0)),
            pl.BlockSpec((1, H), lambda n: (0, 0)),
            pl.BlockSpec((1, H), lambda n: (0, 0)),
            pl.BlockSpec((1, 1), lambda n: (0, 0)),
            pl.BlockSpec((H, H), lambda n: (0, 0)),
            pl.BlockSpec((1, H), lambda n: (0, 0)),
            pl.BlockSpec((H, OUT), lambda n: (0, 0)),
            pl.BlockSpec((1, OUT), lambda n: (0, 0)),
        ],
        out_specs=pl.BlockSpec((G, OUT), lambda n: (0, 0)),
        out_shape=jax.ShapeDtypeStruct((G, OUT), jnp.float32),
        scratch_shapes=[pltpu.VMEM((G, H), jnp.float32)],
    )(h, parts, bidx3, w_sd[:H], w_sd[H:], b_sd.reshape(1, H),
      w_g1, b_g1.reshape(1, H), w_g2.reshape(1, H), b_g2.reshape(1, 1),
      w_m1, b_m1.reshape(1, H), w_m2, b_m2.reshape(1, OUT))


def kernel(x, edge_index, batch_idx, W_nt, b_nt, W_attn, b_attn, W_sd, b_sd,
           W_g1, b_g1, W_g2, b_g2, W_m1, b_m1, W_m2, b_m2):
    src = edge_index[0].astype(jnp.int32)
    dst = edge_index[1].astype(jnp.int32)
    e = src.shape[0]
    grp = NW * CH
    e_pad = ((e + grp - 1) // grp) * grp
    src_pad = jnp.concatenate(
        [src, jnp.full((e_pad - e,), TRASH, jnp.int32)])
    dst_pad = jnp.concatenate([dst, jnp.zeros((e_pad - e,), jnp.int32)])
    dst_all = (dst_pad[None, :]
               + (jnp.arange(T - 1, dtype=jnp.int32) * N)[:, None])
    dst1d = dst_all.reshape(-1)
    zeros = jnp.zeros((ZROWS, H), jnp.float32)

    h, aw = _tc_front(x, W_nt, b_nt, W_attn, b_attn)
    parts = _sc_segsum(aw.reshape((T - 1) * N, H), dst1d, src_pad, zeros)
    return _tc_back(h, parts, batch_idx.astype(jnp.int32).reshape(N // 400, 1, 400),
                    W_sd, b_sd, W_g1, b_g1, W_g2, b_g2, W_m1, b_m1, W_m2, b_m2)
